# baseline (speedup 1.0000x reference)
import functools

import jax
import jax.numpy as jnp
from jax import lax
from jax.experimental import pallas as pl
from jax.experimental.pallas import tpu as pltpu
from jax.experimental.pallas import tpu_sc as plsc

NC = 2
NS = 16
NW = NC * NS
L = 16

def _rsqrt(x):
    i = lax.bitcast_convert_type(x, jnp.int32)
    i = jnp.int32(0x5F3759DF) - lax.shift_right_arithmetic(i, jnp.int32(1))
    y = lax.bitcast_convert_type(i, jnp.float32)
    xh = x * jnp.float32(0.5)
    for _ in range(3):
        y = y * (jnp.float32(1.5) - xh * y * y)
    return y


def _make_kernel(B, S, V, D):
    assert D == 2 * L
    b_per_w = B // NW
    CB = 2
    K = CB * S
    assert K <= 128
    n_chunks = b_per_w // CB
    mesh = plsc.VectorSubcoreMesh(core_axis_name="c", subcore_axis_name="s")

    @functools.partial(
        pl.kernel,
        out_type=jax.ShapeDtypeStruct((B, D), jnp.float32),
        mesh=mesh,
        compiler_params=pltpu.CompilerParams(
            needs_layout_passes=False, use_tc_tiling_on_sc=False
        ),
        scratch_types=[
            pltpu.VMEM((n_chunks, K), jnp.int32),
            pltpu.VMEM((K, D), jnp.float32),
            pltpu.VMEM((K, D), jnp.float32),
            pltpu.VMEM((K, D), jnp.float32),
            pltpu.VMEM((K, D), jnp.float32),
            pltpu.VMEM((b_per_w, D), jnp.float32),
            pltpu.SemaphoreType.DMA,
            pltpu.SemaphoreType.DMA,
            pltpu.SemaphoreType.DMA,
            pltpu.SemaphoreType.DMA,
        ],
    )
    def k(idx_hbm, table_hbm, out_hbm, idx_v, rows0, rows1, rows2, rows3,
          out_v, sem0, sem1, sem2, sem3):
        wid = lax.axis_index("s") * NC + lax.axis_index("c")
        pltpu.sync_copy(idx_hbm.at[pl.ds(wid * n_chunks, n_chunks)], idx_v)

        def fire(jj, rows, sem):
            pltpu.async_copy(table_hbm.at[idx_v.at[jj]], rows, sem)

        fire(0, rows0, sem0)
        fire(1, rows1, sem1)
        fire(2, rows2, sem2)
        fire(3, rows3, sem3)

        def process(jj, rows, sem):
            pltpu.make_async_copy(table_hbm.at[idx_v.at[jj]], rows, sem).wait()
            lane = lax.broadcasted_iota(jnp.int32, (L,), 0)
            for b2 in range(CB):
                acc0 = jnp.zeros((L,), jnp.float32)
                acc1 = jnp.zeros((L,), jnp.float32)
                for g0 in range(0, S, 8):
                    gn = min(8, S - g0)
                    v0s, v1s = [], []
                    norms = jnp.full((L,), jnp.float32(1.0))
                    for t in range(gn):
                        row = b2 * S + g0 + t
                        v0 = rows[row, pl.ds(0, L)]
                        v1 = rows[row, pl.ds(L, L)]
                        v0s.append(v0)
                        v1s.append(v1)
                        tot = jnp.sum(v0 * v0 + v1 * v1)
                        norms = jnp.where(lane == t, tot, norms)
                    inv16 = _rsqrt(norms)
                    for t in range(gn):
                        iv = inv16[t]
                        acc0 = acc0 + v0s[t] * iv
                        acc1 = acc1 + v1s[t] * iv
                tot = jnp.sum(acc0 * acc0 + acc1 * acc1)
                inv = _rsqrt(tot)
                row_out = jj * CB + b2
                out_v[row_out, pl.ds(0, L)] = acc0 * inv
                out_v[row_out, pl.ds(L, L)] = acc1 * inv
            @pl.when(jj < n_chunks - 4)
            def _():
                fire(jj + 4, rows, sem)

        def body(i, _):
            j = i * 4
            process(j, rows0, sem0)
            process(j + 1, rows1, sem1)
            process(j + 2, rows2, sem2)
            process(j + 3, rows3, sem3)
            return 0

        lax.fori_loop(0, n_chunks // 4, body, 0)
        pltpu.sync_copy(out_v, out_hbm.at[pl.ds(wid * b_per_w, b_per_w)])

    return k


def kernel(indices, table):
    B, S = indices.shape
    V, D = table.shape
    k = _make_kernel(B, S, V, D)
    CB = 2
    idx2d = indices.astype(jnp.int32).reshape(B // CB, CB * S)
    return k(idx2d, table)


# speedup vs baseline: 1.1538x; 1.1538x over previous
import functools

import jax
import jax.numpy as jnp
from jax import lax
from jax.experimental import pallas as pl
from jax.experimental.pallas import tpu as pltpu
from jax.experimental.pallas import tpu_sc as plsc

NC = 2
NS = 16
NW = NC * NS
L = 16
NBUF = 8


def _rsqrt(x):
    i = lax.bitcast_convert_type(x, jnp.int32)
    i = jnp.int32(0x5F3759DF) - lax.shift_right_arithmetic(i, jnp.int32(1))
    y = lax.bitcast_convert_type(i, jnp.float32)
    xh = x * jnp.float32(0.5)
    for _ in range(3):
        y = y * (jnp.float32(1.5) - xh * y * y)
    return y


def _make_kernel(B, S, V, D):
    assert D == 2 * L
    b_per_w = B // NW
    CB = 2
    K = CB * S
    assert K <= 128
    n_chunks = b_per_w // CB
    assert n_chunks % NBUF == 0
    mesh = plsc.VectorSubcoreMesh(core_axis_name="c", subcore_axis_name="s")

    @functools.partial(
        pl.kernel,
        out_type=jax.ShapeDtypeStruct((B, D), jnp.float32),
        mesh=mesh,
        compiler_params=pltpu.CompilerParams(
            needs_layout_passes=False, use_tc_tiling_on_sc=False
        ),
        scratch_types=[
            pltpu.VMEM((n_chunks, K), jnp.int32),
            pltpu.VMEM((NBUF * K, D), jnp.float32),
            pltpu.VMEM((b_per_w, D), jnp.float32),
        ]
        + [pltpu.SemaphoreType.DMA] * NBUF,
    )
    def k(idx_hbm, table_hbm, out_hbm, idx_v, rows, out_v, *sems):
        wid = lax.axis_index("s") * NC + lax.axis_index("c")
        pltpu.sync_copy(idx_hbm.at[pl.ds(wid * n_chunks, n_chunks)], idx_v)

        def fire(jj, s):
            pltpu.async_copy(
                table_hbm.at[idx_v.at[jj]],
                rows.at[pl.ds(s * K, K)],
                sems[s],
            )

        for s in range(NBUF):
            fire(s, s)

        lane = lax.broadcasted_iota(jnp.int32, (L,), 0)

        def body(jj, _):
            slot = lax.rem(jj, jnp.int32(NBUF))
            for s in range(NBUF):
                @pl.when(slot == s)
                def _():
                    pltpu.make_async_copy(
                        table_hbm.at[idx_v.at[jj]],
                        rows.at[pl.ds(s * K, K)],
                        sems[s],
                    ).wait()

                    @pl.when(jj < n_chunks - NBUF)
                    def _():
                        fire(jj + NBUF, s)

            base = slot * K
            for b2 in range(CB):
                acc0 = jnp.zeros((L,), jnp.float32)
                acc1 = jnp.zeros((L,), jnp.float32)
                for g0 in range(0, S, 8):
                    gn = min(8, S - g0)
                    v0s, v1s = [], []
                    norms = jnp.full((L,), jnp.float32(1.0))
                    for t in range(gn):
                        row = base + b2 * S + g0 + t
                        v0 = rows[row, pl.ds(0, L)]
                        v1 = rows[row, pl.ds(L, L)]
                        v0s.append(v0)
                        v1s.append(v1)
                        tot = jnp.sum(v0 * v0 + v1 * v1)
                        norms = jnp.where(lane == t, tot, norms)
                    inv16 = _rsqrt(norms)
                    for t in range(gn):
                        iv = inv16[t]
                        acc0 = acc0 + v0s[t] * iv
                        acc1 = acc1 + v1s[t] * iv
                tot = jnp.sum(acc0 * acc0 + acc1 * acc1)
                inv = _rsqrt(tot)
                row_out = jj * CB + b2
                out_v[row_out, pl.ds(0, L)] = acc0 * inv
                out_v[row_out, pl.ds(L, L)] = acc1 * inv
            return 0

        lax.fori_loop(0, n_chunks, body, 0)
        pltpu.sync_copy(out_v, out_hbm.at[pl.ds(wid * b_per_w, b_per_w)])

    return k


def kernel(indices, table):
    B, S = indices.shape
    V, D = table.shape
    k = _make_kernel(B, S, V, D)
    CB = 2
    idx2d = indices.astype(jnp.int32).reshape(B // CB, CB * S)
    return k(idx2d, table)
